# Initial kernel scaffold; baseline (speedup 1.0000x reference)
import functools

import jax
import jax.numpy as jnp
from jax import lax
from jax.experimental import pallas as pl
from jax.experimental.pallas import tpu as pltpu
from jax.experimental.pallas import tpu_sc as plsc

N = 50000
D = 128
G = 1024
GP = 1056
SC_COLS = 144
BT_COLS = 144
SUB = 80
ROWS_PER_SUBCORE = GP // 16
F32 = jnp.float32

_mesh = plsc.VectorSubcoreMesh(core_axis_name="c", subcore_axis_name="s")


def _worker_layout():
    cid = lax.axis_index("c")
    sid = lax.axis_index("s")
    wid = sid * 2 + cid
    base = jnp.where(wid < 17, wid * 1600, 27200 + (wid - 17) * 1520)
    nsub = jnp.where(wid < 17, 20, 19)
    return cid, sid, base, nsub


def _zero_acc(zeros_hbm, acc_sh, sid):
    r0 = sid * ROWS_PER_SUBCORE
    pltpu.sync_copy(zeros_hbm.at[pl.ds(r0, ROWS_PER_SUBCORE)],
                    acc_sh.at[pl.ds(r0, ROWS_PER_SUBCORE)])
    plsc.subcore_barrier()


def _flush_acc(acc_sh, out_hbm, cid, sid):
    plsc.subcore_barrier()
    r0 = sid * ROWS_PER_SUBCORE
    pltpu.sync_copy(acc_sh.at[pl.ds(r0, ROWS_PER_SUBCORE)],
                    out_hbm.at[cid, pl.ds(r0, ROWS_PER_SUBCORE)])


def _sc_segsum_body(z_hbm, gid_hbm, zeros_hbm, out_hbm, idx_v, rows_v, acc_sh, sem):
    cid, sid, base, nsub = _worker_layout()
    _zero_acc(zeros_hbm, acc_sh, sid)

    def step(j, _):
        off = base + j * SUB
        pltpu.sync_copy(gid_hbm.at[pl.ds(off, SUB)], idx_v)
        cp = pltpu.async_copy(z_hbm.at[pl.ds(off, SUB)], rows_v, sem)
        cp.wait()
        pltpu.sync_copy(rows_v, acc_sh.at[idx_v], add=True)
        return 0

    lax.fori_loop(0, nsub, step, 0)
    _flush_acc(acc_sh, out_hbm, cid, sid)


def _sc_segsum(z, gid, zeros128):
    return pl.kernel(
        _sc_segsum_body,
        out_type=jax.ShapeDtypeStruct((2, GP, D), F32),
        mesh=_mesh,
        scratch_types=[
            pltpu.VMEM((SUB,), jnp.int32),
            pltpu.VMEM((SUB, D), F32),
            pltpu.VMEM_SHARED((GP, D), F32),
            pltpu.SemaphoreType.DMA,
        ],
    )(z, gid, zeros128)


def _rsqrt_scalar(q):
    qi = lax.bitcast_convert_type(q, jnp.int32)
    yi = jnp.int32(0x5F3759DF) - (qi >> 1)
    y = lax.bitcast_convert_type(yi, F32)
    for _ in range(3):
        y = y * (1.5 - 0.5 * q * y * y)
    return y


def _sc_pass_c_body(write_u, h_hbm, gid_hbm, bt_hbm, ut_hbm, zeros_hbm,
                    s_out, u_out, idx_v, h_v, rows_v, u_v, out_v, acc_sh,
                    sem, sem2):
    cid, sid, base, nsub = _worker_layout()
    _zero_acc(zeros_hbm, acc_sh, sid)
    mask0 = (lax.iota(jnp.int32, 16) == 0).astype(F32)

    def step(j, _):
        off = base + j * SUB
        pltpu.sync_copy(gid_hbm.at[pl.ds(off, SUB)], idx_v)
        pltpu.sync_copy(h_hbm.at[pl.ds(off, SUB)], h_v)
        pltpu.async_copy(bt_hbm.at[idx_v], rows_v, sem).wait()
        if write_u:
            cp_u = pltpu.async_copy(ut_hbm.at[idx_v], u_v, sem2)

        def atom(a, _):
            hks = [h_v[a, k * 16:(k + 1) * 16] for k in range(8)]
            bks = [rows_v[a, k * 16:(k + 1) * 16] for k in range(8)]
            acc_d = hks[0] * bks[0]
            acc_q = hks[0] * hks[0]
            for k in range(1, 8):
                acc_d = acc_d + hks[k] * bks[k]
                acc_q = acc_q + hks[k] * hks[k]
            dot = jnp.sum(acc_d)
            q = jnp.sum(acc_q)
            hn = q * _rsqrt_scalar(q)
            bn = rows_v[a, 128]
            den = jnp.maximum(hn * bn, 1e-8)
            arg = dot / den - 1.0
            ev = jnp.exp(jnp.full((16,), arg, dtype=F32))
            for k in range(8):
                out_v[a, k * 16:(k + 1) * 16] = ev * hks[k]
            out_v[a, 128:144] = ev * mask0
            return 0

        lax.fori_loop(0, SUB, atom, 0)
        pltpu.sync_copy(out_v, acc_sh.at[idx_v], add=True)
        if write_u:
            cp_u.wait()
            pltpu.sync_copy(u_v, u_out.at[pl.ds(off, SUB)])
        return 0

    lax.fori_loop(0, nsub, step, 0)
    _flush_acc(acc_sh, s_out, cid, sid)


def _sc_pass_c(h, gid, bt, zeros144):
    def wrapped(h_hbm, gid_hbm, bt_hbm, zeros_hbm, s_out, idx_v, h_v, rows_v,
                out_v, acc_sh, sem):
        _sc_pass_c_body(False, h_hbm, gid_hbm, bt_hbm, None, zeros_hbm, s_out,
                        None, idx_v, h_v, rows_v, None, out_v, acc_sh, sem,
                        None)

    return pl.kernel(
        wrapped,
        out_type=jax.ShapeDtypeStruct((2, GP, SC_COLS), F32),
        mesh=_mesh,
        scratch_types=[
            pltpu.VMEM((SUB,), jnp.int32),
            pltpu.VMEM((SUB, D), F32),
            pltpu.VMEM((SUB, BT_COLS), F32),
            pltpu.VMEM((SUB, SC_COLS), F32),
            pltpu.VMEM_SHARED((GP, SC_COLS), F32),
            pltpu.SemaphoreType.DMA,
        ],
    )(h, gid, bt, zeros144)


def _sc_pass_c_u(h, gid, bt, ut, zeros144):
    body = functools.partial(_sc_pass_c_body, True)
    return pl.kernel(
        body,
        out_type=(jax.ShapeDtypeStruct((2, GP, SC_COLS), F32),
                  jax.ShapeDtypeStruct((N, D), F32)),
        mesh=_mesh,
        scratch_types=[
            pltpu.VMEM((SUB,), jnp.int32),
            pltpu.VMEM((SUB, D), F32),
            pltpu.VMEM((SUB, BT_COLS), F32),
            pltpu.VMEM((SUB, D), F32),
            pltpu.VMEM((SUB, SC_COLS), F32),
            pltpu.VMEM_SHARED((GP, SC_COLS), F32),
            pltpu.SemaphoreType.DMA,
            pltpu.SemaphoreType.DMA,
        ],
    )(h, gid, bt, ut, zeros144)


def _sc_gather_body(tab_hbm, gid_hbm, out_hbm, idx_v, rows_v, sem):
    _, _, base, nsub = _worker_layout()

    def step(j, _):
        off = base + j * SUB
        pltpu.sync_copy(gid_hbm.at[pl.ds(off, SUB)], idx_v)
        pltpu.async_copy(tab_hbm.at[idx_v], rows_v, sem).wait()
        pltpu.sync_copy(rows_v, out_hbm.at[pl.ds(off, SUB)])
        return 0

    lax.fori_loop(0, nsub, step, 0)


def _sc_gather(tab, gid):
    return pl.kernel(
        _sc_gather_body,
        out_type=jax.ShapeDtypeStruct((N, D), F32),
        mesh=_mesh,
        scratch_types=[
            pltpu.VMEM((SUB,), jnp.int32),
            pltpu.VMEM((SUB, D), F32),
            pltpu.SemaphoreType.DMA,
        ],
    )(tab, gid)


def _leaky(x):
    return jnp.where(x >= 0, x, 0.01 * x)


def _dot(a, b):
    return jnp.dot(a, b, preferred_element_type=F32)


def _tc_d0_body(p_ref, obt_ref):
    br = p_ref[0, :G, :] + p_ref[1, :G, :]
    bn = jnp.sqrt(jnp.sum(br * br, axis=1, keepdims=True))
    obt_ref[...] = jnp.concatenate(
        [br, bn, jnp.zeros((G, BT_COLS - D - 1), F32)], axis=1)


def _tc_d0(partials):
    return pl.pallas_call(
        _tc_d0_body,
        out_shape=jax.ShapeDtypeStruct((G, BT_COLS), F32),
    )(partials)


def _tc_group_body(s_ref, bt_ref, wmsgT, bmsg, wgbT, wguT, bg, wihbT, whhbT,
                   bihb, bhhb, obt_ref, ou_ref):
    S = s_ref[0, :G, :] + s_ref[1, :G, :]
    Se = S[:, 128:129]
    Seh = S[:, :128]
    br = bt_ref[:, :128]
    Sem = _dot(Seh, wmsgT[...]) + Se * bmsg[...]
    u = _leaky(Sem / jnp.maximum(Se, 1e-9))
    g = jax.nn.sigmoid(_dot(br, wgbT[...]) + _dot(u, wguT[...]) + bg[...])
    bwg = (1.0 - g) * u + g * br
    gi = _dot(u, wihbT[...]) + bihb[...]
    gh = _dot(bwg, whhbT[...]) + bhhb[...]
    r = jax.nn.sigmoid(gi[:, :128] + gh[:, :128])
    zz = jax.nn.sigmoid(gi[:, 128:256] + gh[:, 128:256])
    n = jnp.tanh(gi[:, 256:] + r * gh[:, 256:])
    bnew = (1.0 - zz) * n + zz * bwg
    ub = _leaky(_dot(bnew, wmsgT[...]) + bmsg[...])
    bn = jnp.sqrt(jnp.sum(bnew * bnew, axis=1, keepdims=True))
    obt_ref[...] = jnp.concatenate(
        [bnew, bn, jnp.zeros((G, BT_COLS - D - 1), F32)], axis=1)
    ou_ref[...] = ub


def _tc_group(s, bt, wmsgT, bmsg, wgbT, wguT, bg, wihbT, whhbT, bihb, bhhb):
    return pl.pallas_call(
        _tc_group_body,
        out_shape=(jax.ShapeDtypeStruct((G, BT_COLS), F32),
                   jax.ShapeDtypeStruct((G, D), F32)),
    )(s, bt, wmsgT, bmsg, wgbT, wguT, bg, wihbT, whhbT, bihb, bhhb)


def _tc_b2u_body(p_ref, wmsgT, bmsg, ou_ref):
    br = p_ref[0, :G, :] + p_ref[1, :G, :]
    ou_ref[...] = _leaky(_dot(br, wmsgT[...]) + bmsg[...])


def _tc_b2u(partials, wmsgT, bmsg):
    return pl.pallas_call(
        _tc_b2u_body,
        out_shape=jax.ShapeDtypeStruct((G, D), F32),
    )(partials, wmsgT, bmsg)


_BLK = 2000


def _gru_step(gi, gh, h):
    r = jax.nn.sigmoid(gi[:, :128] + gh[:, :128])
    zz = jax.nn.sigmoid(gi[:, 128:256] + gh[:, 128:256])
    n = jnp.tanh(gi[:, 256:] + r * gh[:, 256:])
    return (1.0 - zz) * n + zz * h


def _tc_zupdate_body(z_ref, u1_ref, u2_ref, u3_ref, wgbT, wguT, bg, wihT,
                     whhT, bih, bhh, o_ref):
    Z = z_ref[...]
    for u_ref in (u1_ref, u2_ref, u3_ref):
        u = u_ref[...]
        g = jax.nn.sigmoid(_dot(Z, wgbT[...]) + _dot(u, wguT[...]) + bg[...])
        m = (1.0 - g) * u + g * Z
        gi = _dot(m, wihT[...]) + bih[...]
        gh = _dot(Z, whhT[...]) + bhh[...]
        Z = _gru_step(gi, gh, Z)
    o_ref[...] = Z


def _tc_zupdate(z, u1, u2, u3, wgbT, wguT, bg, wihT, whhT, bih, bhh):
    row = lambda i: (i, 0)
    rep = lambda i: (0, 0)
    return pl.pallas_call(
        _tc_zupdate_body,
        grid=(N // _BLK,),
        in_specs=[
            pl.BlockSpec((_BLK, D), row),
            pl.BlockSpec((_BLK, D), row),
            pl.BlockSpec((_BLK, D), row),
            pl.BlockSpec((_BLK, D), row),
            pl.BlockSpec((D, D), rep),
            pl.BlockSpec((D, D), rep),
            pl.BlockSpec((1, D), rep),
            pl.BlockSpec((D, 3 * D), rep),
            pl.BlockSpec((D, 3 * D), rep),
            pl.BlockSpec((1, 3 * D), rep),
            pl.BlockSpec((1, 3 * D), rep),
        ],
        out_specs=pl.BlockSpec((_BLK, D), row),
        out_shape=jax.ShapeDtypeStruct((N, D), F32),
    )(z, u1, u2, u3, wgbT, wguT, bg, wihT, whhT, bih, bhh)


def _tc_hupdate_body(h_ref, uh_ref, wihT, whhT, bih, bhh, o_ref):
    Hb = h_ref[...]
    gi = _dot(uh_ref[...], wihT[...]) + bih[...]
    for _ in range(3):
        gh = _dot(Hb, whhT[...]) + bhh[...]
        Hb = _gru_step(gi, gh, Hb)
    o_ref[...] = Hb


def _tc_hupdate(h, uh, wihT, whhT, bih, bhh):
    row = lambda i: (i, 0)
    rep = lambda i: (0, 0)
    return pl.pallas_call(
        _tc_hupdate_body,
        grid=(N // _BLK,),
        in_specs=[
            pl.BlockSpec((_BLK, D), row),
            pl.BlockSpec((_BLK, D), row),
            pl.BlockSpec((D, 3 * D), rep),
            pl.BlockSpec((D, 3 * D), rep),
            pl.BlockSpec((1, 3 * D), rep),
            pl.BlockSpec((1, 3 * D), rep),
        ],
        out_specs=pl.BlockSpec((_BLK, D), row),
        out_shape=jax.ShapeDtypeStruct((N, D), F32),
    )(h, uh, wihT, whhT, bih, bhh)


def kernel(H_intra, Z_inter, group_assign, W_msg, b_msg, W_gB, b_gB, W_gu,
           b_gu, W_ih_b, W_hh_b, b_ih_b, b_hh_b, W_ih_a, W_hh_a, b_ih_a,
           b_hh_a):
    gid = group_assign.astype(jnp.int32)
    zeros128 = jnp.zeros((GP, D), F32)
    zeros144 = jnp.zeros((GP, SC_COLS), F32)

    wmsgT = W_msg.T
    wgbT = W_gB.T
    wguT = W_gu.T
    wihbT = W_ih_b.T
    whhbT = W_hh_b.T
    wihaT = W_ih_a.T
    whhaT = W_hh_a.T
    bmsg = b_msg.reshape(1, D)
    bg = (b_gB + b_gu).reshape(1, D)
    bihb = b_ih_b.reshape(1, 3 * D)
    bhhb = b_hh_b.reshape(1, 3 * D)
    biha = b_ih_a.reshape(1, 3 * D)
    bhha = b_hh_a.reshape(1, 3 * D)

    bridge_p = _sc_segsum(Z_inter, gid, zeros128)
    bt = _tc_d0(bridge_p)

    s1 = _sc_pass_c(H_intra, gid, bt, zeros144)
    bt, ut1 = _tc_group(s1, bt, wmsgT, bmsg, wgbT, wguT, bg, wihbT, whhbT,
                        bihb, bhhb)
    s2, u1 = _sc_pass_c_u(H_intra, gid, bt, ut1, zeros144)
    bt, ut2 = _tc_group(s2, bt, wmsgT, bmsg, wgbT, wguT, bg, wihbT, whhbT,
                        bihb, bhhb)
    s3, u2 = _sc_pass_c_u(H_intra, gid, bt, ut2, zeros144)
    _, ut3 = _tc_group(s3, bt, wmsgT, bmsg, wgbT, wguT, bg, wihbT, whhbT,
                       bihb, bhhb)
    u3 = _sc_gather(ut3, gid)

    z_final = _tc_zupdate(Z_inter, u1, u2, u3, wgbT, wguT, bg, wihaT, whhaT,
                          biha, bhha)

    b2_p = _sc_segsum(z_final, gid, zeros128)
    uh_t = _tc_b2u(b2_p, wmsgT, bmsg)
    uh = _sc_gather(uh_t, gid)
    h_final = _tc_hupdate(H_intra, uh, wihaT, whhaT, biha, bhha)

    return (z_final, h_final)


# speedup vs baseline: 3.2604x; 3.2604x over previous
import functools

import jax
import jax.numpy as jnp
from jax import lax
from jax.experimental import pallas as pl
from jax.experimental.pallas import tpu as pltpu
from jax.experimental.pallas import tpu_sc as plsc

N = 50000
D = 128
G = 1024
GP = 1152
SUB = 80
ROWS_PER_SUBCORE = GP // 16
F32 = jnp.float32

_mesh = plsc.VectorSubcoreMesh(core_axis_name="c", subcore_axis_name="s")


def _worker_layout():
    cid = lax.axis_index("c")
    sid = lax.axis_index("s")
    wid = sid * 2 + cid
    base = jnp.where(wid < 17, wid * 1600, 27200 + (wid - 17) * 1520)
    nsub = jnp.where(wid < 17, 20, 19)
    return cid, sid, base, nsub


def _sc_segsum_body(rows_hbm, gid_hbm, zeros_hbm, out_hbm, idx_v, rows_v,
                    acc_sh, sem):
    cid, sid, base, nsub = _worker_layout()
    r0 = sid * ROWS_PER_SUBCORE
    pltpu.sync_copy(zeros_hbm.at[pl.ds(r0, ROWS_PER_SUBCORE)],
                    acc_sh.at[pl.ds(r0, ROWS_PER_SUBCORE)])
    plsc.subcore_barrier()

    def step(j, _):
        off = base + j * SUB
        pltpu.sync_copy(gid_hbm.at[pl.ds(off, SUB)], idx_v)
        pltpu.async_copy(rows_hbm.at[pl.ds(off, SUB)], rows_v, sem).wait()
        pltpu.sync_copy(rows_v, acc_sh.at[idx_v], add=True)
        return 0

    lax.fori_loop(0, nsub, step, 0)
    plsc.subcore_barrier()
    pltpu.sync_copy(acc_sh.at[pl.ds(r0, ROWS_PER_SUBCORE)],
                    out_hbm.at[cid, pl.ds(r0, ROWS_PER_SUBCORE)])


@functools.lru_cache(maxsize=None)
def _sc_segsum_call(ncols):
    return pl.kernel(
        _sc_segsum_body,
        out_type=jax.ShapeDtypeStruct((2, GP, ncols), F32),
        mesh=_mesh,
        scratch_types=[
            pltpu.VMEM((SUB,), jnp.int32),
            pltpu.VMEM((SUB, ncols), F32),
            pltpu.VMEM_SHARED((GP, ncols), F32),
            pltpu.SemaphoreType.DMA,
        ],
    )


def _sc_segsum(rows, gid, zeros):
    return _sc_segsum_call(rows.shape[1])(rows, gid, zeros)


def _sc_segsum2_body(rh_hbm, re_hbm, gid_hbm, zeros_hbm, ph_hbm, pe_hbm,
                     idx_v, rh_v, re_v, acch_sh, acce_sh, sem):
    cid, sid, base, nsub = _worker_layout()
    r0 = sid * ROWS_PER_SUBCORE
    pltpu.sync_copy(zeros_hbm.at[pl.ds(r0, ROWS_PER_SUBCORE)],
                    acch_sh.at[pl.ds(r0, ROWS_PER_SUBCORE)])
    pltpu.sync_copy(zeros_hbm.at[pl.ds(r0, ROWS_PER_SUBCORE)],
                    acce_sh.at[pl.ds(r0, ROWS_PER_SUBCORE)])
    plsc.subcore_barrier()

    def step(j, _):
        off = base + j * SUB
        pltpu.sync_copy(gid_hbm.at[pl.ds(off, SUB)], idx_v)
        cp = pltpu.async_copy(rh_hbm.at[pl.ds(off, SUB)], rh_v, sem)
        pltpu.sync_copy(re_hbm.at[pl.ds(off, SUB)], re_v)
        cp.wait()
        pltpu.sync_copy(rh_v, acch_sh.at[idx_v], add=True)
        pltpu.sync_copy(re_v, acce_sh.at[idx_v], add=True)
        return 0

    lax.fori_loop(0, nsub, step, 0)
    plsc.subcore_barrier()
    pltpu.sync_copy(acch_sh.at[pl.ds(r0, ROWS_PER_SUBCORE)],
                    ph_hbm.at[cid, pl.ds(r0, ROWS_PER_SUBCORE)])
    pltpu.sync_copy(acce_sh.at[pl.ds(r0, ROWS_PER_SUBCORE)],
                    pe_hbm.at[cid, pl.ds(r0, ROWS_PER_SUBCORE)])


_sc_segsum2 = pl.kernel(
    _sc_segsum2_body,
    out_type=(jax.ShapeDtypeStruct((2, GP, D), F32),
              jax.ShapeDtypeStruct((2, GP, D), F32)),
    mesh=_mesh,
    scratch_types=[
        pltpu.VMEM((SUB,), jnp.int32),
        pltpu.VMEM((SUB, D), F32),
        pltpu.VMEM((SUB, D), F32),
        pltpu.VMEM_SHARED((GP, D), F32),
        pltpu.VMEM_SHARED((GP, D), F32),
        pltpu.SemaphoreType.DMA,
    ],
)


def _sc_gather_body(tab_hbm, gid_hbm, out_hbm, idx_v, rows_v, sem):
    _, _, base, nsub = _worker_layout()

    def step(j, _):
        off = base + j * SUB
        pltpu.sync_copy(gid_hbm.at[pl.ds(off, SUB)], idx_v)
        pltpu.async_copy(tab_hbm.at[idx_v], rows_v, sem).wait()
        pltpu.sync_copy(rows_v, out_hbm.at[pl.ds(off, SUB)])
        return 0

    lax.fori_loop(0, nsub, step, 0)


@functools.lru_cache(maxsize=None)
def _sc_gather_call(ncols):
    return pl.kernel(
        _sc_gather_body,
        out_type=jax.ShapeDtypeStruct((N, ncols), F32),
        mesh=_mesh,
        scratch_types=[
            pltpu.VMEM((SUB,), jnp.int32),
            pltpu.VMEM((SUB, ncols), F32),
            pltpu.SemaphoreType.DMA,
        ],
    )


def _sc_gather(tab, gid):
    return _sc_gather_call(tab.shape[1])(tab, gid)


def _leaky(x):
    return jnp.where(x >= 0, x, 0.01 * x)


def _dot(a, b):
    return jnp.dot(a, b, preferred_element_type=F32)


def _gru_step(gi, gh, h):
    r = jax.nn.sigmoid(gi[:, :D] + gh[:, :D])
    zz = jax.nn.sigmoid(gi[:, D:2 * D] + gh[:, D:2 * D])
    n = jnp.tanh(gi[:, 2 * D:] + r * gh[:, 2 * D:])
    return (1.0 - zz) * n + zz * h


def _tc_d0_body(p_ref, ot_ref):
    ot_ref[...] = p_ref[0, :G, :] + p_ref[1, :G, :]


def _tc_d0(partials):
    return pl.pallas_call(
        _tc_d0_body,
        out_shape=jax.ShapeDtypeStruct((G, D), F32),
    )(partials)


def _tc_group_body(full, sh_ref, se_ref, t_ref, wmsgT, bmsg, wgbT, wguT, bg,
                   wihbT, whhbT, bihb, bhhb, ot_ref):
    Sem = sh_ref[0, :G, :] + sh_ref[1, :G, :]
    Se = se_ref[0, :G, :1] + se_ref[1, :G, :1]
    br = t_ref[:, :D]
    u = _leaky(Sem / jnp.maximum(Se, 1e-9))
    g = jax.nn.sigmoid(_dot(br, wgbT[...]) + _dot(u, wguT[...]) + bg[...])
    bwg = (1.0 - g) * u + g * br
    gi = _dot(u, wihbT[...]) + bihb[...]
    gh = _dot(bwg, whhbT[...]) + bhhb[...]
    bnew = _gru_step(gi, gh, bwg)
    ub = _leaky(_dot(bnew, wmsgT[...]) + bmsg[...])
    if full:
        ot_ref[...] = jnp.concatenate([bnew, ub], axis=1)
    else:
        ot_ref[...] = ub


def _tc_group(full, sh, se, t, *weights):
    ncols = 2 * D if full else D
    return pl.pallas_call(
        functools.partial(_tc_group_body, full),
        out_shape=jax.ShapeDtypeStruct((G, ncols), F32),
    )(sh, se, t, *weights)


def _tc_b2u_body(p_ref, wmsgT, bmsg, ou_ref):
    br = p_ref[0, :G, :] + p_ref[1, :G, :]
    ou_ref[...] = _leaky(_dot(br, wmsgT[...]) + bmsg[...])


def _tc_b2u(partials, wmsgT, bmsg):
    return pl.pallas_call(
        _tc_b2u_body,
        out_shape=jax.ShapeDtypeStruct((G, D), F32),
    )(partials, wmsgT, bmsg)


_BLK = 2000
_row = lambda i: (i, 0)
_rep = lambda i: (0, 0)


def _tc_passa_body(h_ref, g_ref, wmsgT, bmsg, ewh_ref, ewe_ref):
    Hb = h_ref[...]
    Bp = g_ref[...]
    dot = jnp.sum(Hb * Bp, axis=1, keepdims=True)
    hn = jnp.sqrt(jnp.sum(Hb * Hb, axis=1, keepdims=True))
    bn = jnp.sqrt(jnp.sum(Bp * Bp, axis=1, keepdims=True))
    cos = dot / jnp.maximum(hn * bn, 1e-8)
    e = jnp.exp(cos - 1.0)
    msg = _dot(Hb, wmsgT[...]) + bmsg[...]
    ewh_ref[...] = e * msg
    ewe_ref[...] = jnp.concatenate(
        [e, jnp.zeros((_BLK, D - 1), F32)], axis=1)


def _tc_passa(h, g, wmsgT, bmsg):
    return pl.pallas_call(
        _tc_passa_body,
        grid=(N // _BLK,),
        in_specs=[
            pl.BlockSpec((_BLK, D), _row),
            pl.BlockSpec((_BLK, D), _row),
            pl.BlockSpec((D, D), _rep),
            pl.BlockSpec((1, D), _rep),
        ],
        out_specs=(pl.BlockSpec((_BLK, D), _row),
                   pl.BlockSpec((_BLK, D), _row)),
        out_shape=(jax.ShapeDtypeStruct((N, D), F32),
                   jax.ShapeDtypeStruct((N, D), F32)),
    )(h, g, wmsgT, bmsg)


def _tc_zupdate_body(z_ref, g2_ref, g3_ref, u3_ref, wgbT, wguT, bg, wihT,
                     whhT, bih, bhh, o_ref):
    Z = z_ref[...]
    for u in (g2_ref[...], g3_ref[...], u3_ref[...]):
        g = jax.nn.sigmoid(_dot(Z, wgbT[...]) + _dot(u, wguT[...]) + bg[...])
        m = (1.0 - g) * u + g * Z
        gi = _dot(m, wihT[...]) + bih[...]
        gh = _dot(Z, whhT[...]) + bhh[...]
        Z = _gru_step(gi, gh, Z)
    o_ref[...] = Z


def _tc_zupdate(z, g2, g3, u3, wgbT, wguT, bg, wihT, whhT, bih, bhh):
    ucol = lambda i: (i, 1)
    return pl.pallas_call(
        _tc_zupdate_body,
        grid=(N // _BLK,),
        in_specs=[
            pl.BlockSpec((_BLK, D), _row),
            pl.BlockSpec((_BLK, D), ucol),
            pl.BlockSpec((_BLK, D), ucol),
            pl.BlockSpec((_BLK, D), _row),
            pl.BlockSpec((D, D), _rep),
            pl.BlockSpec((D, D), _rep),
            pl.BlockSpec((1, D), _rep),
            pl.BlockSpec((D, 3 * D), _rep),
            pl.BlockSpec((D, 3 * D), _rep),
            pl.BlockSpec((1, 3 * D), _rep),
            pl.BlockSpec((1, 3 * D), _rep),
        ],
        out_specs=pl.BlockSpec((_BLK, D), _row),
        out_shape=jax.ShapeDtypeStruct((N, D), F32),
    )(z, g2, g3, u3, wgbT, wguT, bg, wihT, whhT, bih, bhh)


def _tc_hupdate_body(h_ref, uh_ref, wihT, whhT, bih, bhh, o_ref):
    Hb = h_ref[...]
    gi = _dot(uh_ref[...], wihT[...]) + bih[...]
    for _ in range(3):
        gh = _dot(Hb, whhT[...]) + bhh[...]
        Hb = _gru_step(gi, gh, Hb)
    o_ref[...] = Hb


def _tc_hupdate(h, uh, wihT, whhT, bih, bhh):
    return pl.pallas_call(
        _tc_hupdate_body,
        grid=(N // _BLK,),
        in_specs=[
            pl.BlockSpec((_BLK, D), _row),
            pl.BlockSpec((_BLK, D), _row),
            pl.BlockSpec((D, 3 * D), _rep),
            pl.BlockSpec((D, 3 * D), _rep),
            pl.BlockSpec((1, 3 * D), _rep),
            pl.BlockSpec((1, 3 * D), _rep),
        ],
        out_specs=pl.BlockSpec((_BLK, D), _row),
        out_shape=jax.ShapeDtypeStruct((N, D), F32),
    )(h, uh, wihT, whhT, bih, bhh)


def kernel(H_intra, Z_inter, group_assign, W_msg, b_msg, W_gB, b_gB, W_gu,
           b_gu, W_ih_b, W_hh_b, b_ih_b, b_hh_b, W_ih_a, W_hh_a, b_ih_a,
           b_hh_a):
    gid = group_assign.astype(jnp.int32)
    zeros128 = jnp.zeros((GP, D), F32)

    wmsgT = W_msg.T
    wgbT = W_gB.T
    wguT = W_gu.T
    wihbT = W_ih_b.T
    whhbT = W_hh_b.T
    wihaT = W_ih_a.T
    whhaT = W_hh_a.T
    bmsg = b_msg.reshape(1, D)
    bg = (b_gB + b_gu).reshape(1, D)
    bihb = b_ih_b.reshape(1, 3 * D)
    bhhb = b_hh_b.reshape(1, 3 * D)
    biha = b_ih_a.reshape(1, 3 * D)
    bhha = b_hh_a.reshape(1, 3 * D)
    gw = (wmsgT, bmsg, wgbT, wguT, bg, wihbT, whhbT, bihb, bhhb)

    p0 = _sc_segsum(Z_inter, gid, zeros128)
    t1 = _tc_d0(p0)

    g1 = _sc_gather(t1, gid)
    s1h, s1e = _sc_segsum2(*_tc_passa(H_intra, g1, wmsgT, bmsg), gid, zeros128)
    t2 = _tc_group(True, s1h, s1e, t1, *gw)

    g2 = _sc_gather(t2, gid)
    s2h, s2e = _sc_segsum2(*_tc_passa(H_intra, g2, wmsgT, bmsg), gid, zeros128)
    t3 = _tc_group(True, s2h, s2e, t2, *gw)

    g3 = _sc_gather(t3, gid)
    s3h, s3e = _sc_segsum2(*_tc_passa(H_intra, g3, wmsgT, bmsg), gid, zeros128)
    u3t = _tc_group(False, s3h, s3e, t3, *gw)

    u3 = _sc_gather(u3t, gid)
    z_final = _tc_zupdate(Z_inter, g2, g3, u3, wgbT, wguT, bg, wihaT, whhaT,
                          biha, bhha)

    p2 = _sc_segsum(z_final, gid, zeros128)
    uh = _sc_gather(_tc_b2u(p2, wmsgT, bmsg), gid)
    h_final = _tc_hupdate(H_intra, uh, wihaT, whhaT, biha, bhha)

    return (z_final, h_final)
